# Initial kernel scaffold; baseline (speedup 1.0000x reference)
import jax
import jax.numpy as jnp
from jax.experimental import pallas as pl


def kernel(input_ids, word_emb, pos_emb, tok_emb, ln_gamma, ln_beta):
    raise NotImplementedError("write your pallas kernel here")


# speedup vs baseline: 3.4881x; 3.4881x over previous
import functools

import jax
import jax.numpy as jnp
from jax import lax
from jax.experimental import pallas as pl
from jax.experimental.pallas import tpu as pltpu
from jax.experimental.pallas import tpu_sc as plsc

D = 768
S = 512
B = 128
N = S * B
EPS = 1e-12

NC = 2
NS = 16
NW = NC * NS
PER_W = N // NW
CHUNK = 64
NCHUNK = PER_W // CHUNK
NBUF = 2


def _sc_gather(word_emb, ids_1d):
    mesh = plsc.VectorSubcoreMesh(core_axis_name="core", subcore_axis_name="subcore")

    @functools.partial(
        pl.kernel,
        out_type=jax.ShapeDtypeStruct((N, D), jnp.float32),
        mesh=mesh,
        scratch_types=[
            pltpu.VMEM((PER_W,), jnp.int32),
            pltpu.VMEM((NBUF, CHUNK, D), jnp.float32),
            pltpu.SemaphoreType.DMA,
            pltpu.SemaphoreType.DMA,
            pltpu.SemaphoreType.DMA,
            pltpu.SemaphoreType.DMA,
        ],
    )
    def gather_kernel(x_hbm, i_hbm, o_hbm, idx_v, rows_v, g0, g1, s0, s1):
        gsem = (g0, g1)
        ssem = (s0, s1)
        wid = lax.axis_index("subcore") * NC + lax.axis_index("core")
        base = pl.multiple_of(wid * PER_W, PER_W)
        pltpu.sync_copy(i_hbm.at[pl.ds(base, PER_W)], idx_v)

        def fire_gather(c, b):
            return pltpu.async_copy(
                x_hbm.at[idx_v.at[pl.ds(c * CHUNK, CHUNK)]], rows_v.at[b], gsem[b]
            )

        def fire_store(c, b):
            return pltpu.async_copy(
                rows_v.at[b], o_hbm.at[pl.ds(base + c * CHUNK, CHUNK)], ssem[b]
            )

        def wait_store(b):
            pltpu.make_async_copy(
                rows_v.at[b], o_hbm.at[pl.ds(0, CHUNK)], ssem[b]
            ).wait()

        @pl.loop(0, NCHUNK, step=NBUF)
        def _(c0):
            @pl.when(c0 != 0)
            def _():
                wait_store(0)

            g_a = fire_gather(c0, 0)

            @pl.when(c0 != 0)
            def _():
                wait_store(1)

            g_b = fire_gather(c0 + 1, 1)
            g_a.wait()
            fire_store(c0, 0)
            g_b.wait()
            fire_store(c0 + 1, 1)

        wait_store(0)
        wait_store(1)

    return gather_kernel(word_emb, ids_1d)


RB = 8


def _ln_body(x_ref, pos_ref, tok_ref, g_ref, b_ref, o_ref):
    x = x_ref[...]
    bias = pos_ref[...] + tok_ref[...]
    emb = x + bias[:, None, :]
    mean = jnp.mean(emb, axis=-1, keepdims=True)
    c = emb - mean
    var = jnp.mean(c * c, axis=-1, keepdims=True)
    normed = c * lax.rsqrt(var + EPS)
    g = g_ref[...][0]
    b = b_ref[...][0]
    o_ref[...] = normed * g[None, None, :] + b[None, None, :]


def _tc_ln(gathered3, pos_emb, tok_row, gamma2, beta2):
    return pl.pallas_call(
        _ln_body,
        grid=(S // RB,),
        in_specs=[
            pl.BlockSpec((RB, B, D), lambda i: (i, 0, 0)),
            pl.BlockSpec((RB, D), lambda i: (i, 0)),
            pl.BlockSpec((1, D), lambda i: (0, 0)),
            pl.BlockSpec((1, D), lambda i: (0, 0)),
            pl.BlockSpec((1, D), lambda i: (0, 0)),
        ],
        out_specs=pl.BlockSpec((RB, B, D), lambda i: (i, 0, 0)),
        out_shape=jax.ShapeDtypeStruct((S, B, D), jnp.float32),
        compiler_params=pltpu.CompilerParams(
            dimension_semantics=("arbitrary",),
        ),
    )(gathered3, pos_emb, tok_row, gamma2, beta2)


def kernel(input_ids, word_emb, pos_emb, tok_emb, ln_gamma, ln_beta):
    ids1 = input_ids.astype(jnp.int32).reshape(N)
    gathered = _sc_gather(word_emb, ids1)
    g3 = gathered.reshape(S, B, D)
    tok_row = tok_emb[0:1]
    gamma2 = ln_gamma.reshape(1, D)
    beta2 = ln_beta.reshape(1, D)
    return _tc_ln(g3, pos_emb, tok_row, gamma2, beta2)
